# Initial kernel scaffold; baseline (speedup 1.0000x reference)
import jax
import jax.numpy as jnp
from jax.experimental import pallas as pl


def kernel(pred, label):
    raise NotImplementedError("write your pallas kernel here")


# speedup vs baseline: 2.6674x; 2.6674x over previous
import functools

import jax
import jax.numpy as jnp
from jax import lax
from jax.experimental import pallas as pl
from jax.experimental.pallas import tpu as pltpu

_NC = 19
_R = 128
_H = 512
_W = 512
_B = 4


def _iou_from_hist(h):
    ri = lax.broadcasted_iota(jnp.int32, (_NC, _NC), 0)
    ci = lax.broadcasted_iota(jnp.int32, (_NC, _NC), 1)
    eye = ri == ci
    d = jnp.sum(jnp.where(eye, h, 0.0), axis=1)
    row = jnp.sum(h, axis=1)
    col = jnp.sum(h, axis=0)
    denom = row + col - d
    idx = lax.iota(jnp.int32, _NC)
    valid = (denom > 0.0) & (idx >= 1)
    iou = jnp.where(valid, d / jnp.where(denom > 0.0, denom, 1.0), 0.0)
    cnt = jnp.sum(valid.astype(jnp.float32))
    return 1.0 - jnp.sum(iou) / cnt


def _fused_body(pred_ref, label_ref, out_ref, hist_ref):
    b = pl.program_id(0)
    r = pl.program_id(1)
    nr = pl.num_programs(1)

    x = pred_ref[0]
    lab = label_ref[0]

    best = x[0]
    bidx = jnp.zeros((_R, _W), jnp.int32)
    for c in range(1, _NC):
        xc = x[c]
        take = xc > best
        best = jnp.where(take, xc, best)
        bidx = jnp.where(take, c, bidx)

    cls = lax.broadcasted_iota(jnp.int32, (_NC, _R, _W), 0)
    oh_l = (lab[None] == cls).astype(jnp.bfloat16).reshape(_NC, _R * _W)
    oh_p = (bidx[None] == cls).astype(jnp.bfloat16).reshape(_NC, _R * _W)
    contrib = lax.dot_general(
        oh_l, oh_p, (((1,), (1,)), ((), ())),
        preferred_element_type=jnp.float32)

    @pl.when((b == 0) & (r == 0))
    def _init():
        hist_ref[...] = jnp.zeros_like(hist_ref)

    hist_ref[...] += contrib

    @pl.when((b == _B - 1) & (r == nr - 1))
    def _final():
        out_ref[...] = _iou_from_hist(hist_ref[...]).reshape(1, 1)


@functools.partial(jax.jit, static_argnames=())
def kernel(pred, label):
    label = label.astype(jnp.int32)
    out = pl.pallas_call(
        _fused_body,
        grid=(_B, _H // _R),
        in_specs=[
            pl.BlockSpec((1, _NC, _R, _W), lambda b, r: (b, 0, r, 0)),
            pl.BlockSpec((1, _R, _W), lambda b, r: (b, r, 0)),
        ],
        out_specs=pl.BlockSpec((1, 1), lambda b, r: (0, 0)),
        out_shape=jax.ShapeDtypeStruct((1, 1), jnp.float32),
        scratch_shapes=[pltpu.VMEM((_NC, _NC), jnp.float32)],
    )(pred, label)
    return out[0, 0]
